# Initial kernel scaffold; baseline (speedup 1.0000x reference)
import jax
import jax.numpy as jnp
from jax.experimental import pallas as pl


def kernel(x, We1, be1, We2, be2, We3, be3, Wd1, bd1, Wd2, bd2, Wd3, bd3):
    raise NotImplementedError("write your pallas kernel here")


# speedup vs baseline: 5.6593x; 5.6593x over previous
import functools

import jax
import jax.numpy as jnp
from jax.experimental import pallas as pl
from jax.experimental.pallas import tpu as pltpu

K = 15
SLOTS = 16
LAM = 1.0


def _mlp_body(x_ref, We1_ref, be1_ref, We2_ref, be2_ref, We3_ref, be3_ref,
              Wd1_ref, bd1_ref, Wd2_ref, bd2_ref, Wd3_ref, bd3_ref,
              z_ref, xr2_ref, zr2_ref, rec_ref):
    pid = pl.program_id(0)
    x = x_ref[...]
    f32 = jnp.float32
    h = jnp.maximum(jnp.dot(x, We1_ref[...], preferred_element_type=f32)
                    + be1_ref[...], 0.0)
    h = jnp.maximum(jnp.dot(h, We2_ref[...], preferred_element_type=f32)
                    + be2_ref[...], 0.0)
    z = jnp.dot(h, We3_ref[...], preferred_element_type=f32) + be3_ref[...]
    h = jnp.maximum(jnp.dot(z, Wd1_ref[...], preferred_element_type=f32)
                    + bd1_ref[...], 0.0)
    h = jnp.maximum(jnp.dot(h, Wd2_ref[...], preferred_element_type=f32)
                    + bd2_ref[...], 0.0)
    x_rec = jnp.dot(h, Wd3_ref[...], preferred_element_type=f32) + bd3_ref[...]
    z_ref[...] = z
    xr2_ref[...] = jnp.sum(x * x, axis=1, keepdims=True)
    zr2_ref[...] = jnp.sum(z * z, axis=1, keepdims=True)

    @pl.when(pid == 0)
    def _():
        rec_ref[...] = jnp.zeros_like(rec_ref)

    diff = x_rec - x
    rec_ref[...] += jnp.sum(diff * diff).reshape(1, 1)


def _knn_body(bm, x_ref, xT_ref, xr2row_ref, vals_ref, idxs_ref, kth_ref):
    pid = pl.program_id(0)
    x = x_ref[...]
    g = jnp.dot(x, xT_ref[...], preferred_element_type=jnp.float32)
    r_blk = jnp.sum(x * x, axis=1, keepdims=True)
    d2 = jnp.maximum(r_blk + xr2row_ref[...] - 2.0 * g, 0.0)
    col = jax.lax.broadcasted_iota(jnp.int32, d2.shape, 1)
    row = jax.lax.broadcasted_iota(jnp.int32, d2.shape, 0) + pid * bm
    work = jnp.where(col == row, jnp.inf, d2)
    m = None
    for t in range(K):
        m = jnp.min(work, axis=1, keepdims=True)
        idx = jnp.min(jnp.where(work == m, col, jnp.int32(2 ** 30)),
                      axis=1, keepdims=True)
        vals_ref[:, t:t + 1] = m
        idxs_ref[:, t:t + 1] = idx
        work = jnp.where(col == idx, jnp.inf, work)
    vals_ref[:, K:K + 1] = jnp.zeros((bm, 1), jnp.float32)
    idxs_ref[:, K:K + 1] = row[:, :1]
    kth_ref[...] = m


def _pair_body(z_ref, zT_ref, zr2row_ref, idx_ref, kthrow_ref,
               zd2_ref, gk_ref):
    z = z_ref[...]
    g = jnp.dot(z, zT_ref[...], preferred_element_type=jnp.float32)
    zr_blk = jnp.sum(z * z, axis=1, keepdims=True)
    zd2 = jnp.maximum(zr_blk + zr2row_ref[...] - 2.0 * g, 0.0)
    col = jax.lax.broadcasted_iota(jnp.int32, zd2.shape, 1)
    kthr = kthrow_ref[...]
    for s in range(SLOTS):
        sel = col == idx_ref[:, s:s + 1]
        zd2_ref[:, s:s + 1] = jnp.sum(jnp.where(sel, zd2, 0.0),
                                      axis=1, keepdims=True)
        gk_ref[:, s:s + 1] = jnp.sum(jnp.where(sel, kthr, 0.0),
                                     axis=1, keepdims=True)


def _final_body(b, d_in, vals_ref, zd2p_ref, gk_ref, rec_ref,
                tot_ref, rl_ref, dl_ref):
    xd2 = vals_ref[...]
    zd2 = zd2p_ref[...]
    gk = gk_ref[...]
    slot = jax.lax.broadcasted_iota(jnp.int32, xd2.shape, 1) < K
    kth = xd2[:, K - 1:K]
    xmax = jnp.sqrt(jnp.max(kth))
    zmax = jnp.sqrt(jnp.max(jnp.where(slot, zd2, -jnp.inf)))
    xm = xmax + 1e-8
    zm = zmax + 1e-8
    t = (jnp.sqrt(zd2) / zm - jnp.sqrt(xd2) / xm) ** 2
    s2 = jnp.sum(jnp.where(slot, t, 0.0))
    mut = slot & (xd2 <= gk)
    smut = jnp.sum(jnp.where(mut, t, 0.0))
    nmut = jnp.sum(mut.astype(jnp.float32))
    cnt = 2.0 * b * K - nmut
    dl = ((2.0 * s2 - smut) / cnt).reshape(1, 1)
    rl = rec_ref[...] / (b * d_in)
    dl_ref[...] = dl
    rl_ref[...] = rl
    tot_ref[...] = rl + LAM * dl


def kernel(x, We1, be1, We2, be2, We3, be3, Wd1, bd1, Wd2, bd2, Wd3, bd3):
    b, d_in = x.shape
    h1 = We1.shape[1]
    h2 = We2.shape[1]
    lat = We3.shape[1]
    bm = 256 if b % 256 == 0 else b
    nb = b // bm
    f32 = jnp.float32

    full = lambda shape: pl.BlockSpec(shape, lambda i: (0, 0))
    rowblk = lambda w: pl.BlockSpec((bm, w), lambda i: (i, 0))

    z, xr2, zr2, rec = pl.pallas_call(
        _mlp_body,
        grid=(nb,),
        in_specs=[
            rowblk(d_in),
            full((d_in, h1)), full((1, h1)),
            full((h1, h2)), full((1, h2)),
            full((h2, lat)), full((1, lat)),
            full((lat, h2)), full((1, h2)),
            full((h2, h1)), full((1, h1)),
            full((h1, d_in)), full((1, d_in)),
        ],
        out_specs=[
            rowblk(lat), rowblk(1), rowblk(1),
            pl.BlockSpec((1, 1), lambda i: (0, 0)),
        ],
        out_shape=[
            jax.ShapeDtypeStruct((b, lat), f32),
            jax.ShapeDtypeStruct((b, 1), f32),
            jax.ShapeDtypeStruct((b, 1), f32),
            jax.ShapeDtypeStruct((1, 1), f32),
        ],
    )(x, We1, be1.reshape(1, -1), We2, be2.reshape(1, -1),
      We3, be3.reshape(1, -1), Wd1, bd1.reshape(1, -1),
      Wd2, bd2.reshape(1, -1), Wd3, bd3.reshape(1, -1))

    vals, idxs, kth = pl.pallas_call(
        functools.partial(_knn_body, bm),
        grid=(nb,),
        in_specs=[rowblk(d_in), full((d_in, b)), full((1, b))],
        out_specs=[rowblk(SLOTS), rowblk(SLOTS), rowblk(1)],
        out_shape=[
            jax.ShapeDtypeStruct((b, SLOTS), f32),
            jax.ShapeDtypeStruct((b, SLOTS), jnp.int32),
            jax.ShapeDtypeStruct((b, 1), f32),
        ],
    )(x, x.T, xr2.reshape(1, b))

    zd2p, gk = pl.pallas_call(
        _pair_body,
        grid=(nb,),
        in_specs=[rowblk(lat), full((lat, b)), full((1, b)),
                  rowblk(SLOTS), full((1, b))],
        out_specs=[rowblk(SLOTS), rowblk(SLOTS)],
        out_shape=[
            jax.ShapeDtypeStruct((b, SLOTS), f32),
            jax.ShapeDtypeStruct((b, SLOTS), f32),
        ],
    )(z, z.T, zr2.reshape(1, b), idxs, kth.reshape(1, b))

    tot, rl, dl = pl.pallas_call(
        functools.partial(_final_body, b, d_in),
        in_specs=[pl.BlockSpec((b, SLOTS), lambda: (0, 0))] * 3
        + [pl.BlockSpec((1, 1), lambda: (0, 0))],
        out_specs=[pl.BlockSpec((1, 1), lambda: (0, 0))] * 3,
        out_shape=[jax.ShapeDtypeStruct((1, 1), f32)] * 3,
    )(vals, zd2p, gk, rec)

    return (tot[0, 0], rl[0, 0], dl[0, 0])


# speedup vs baseline: 6.3498x; 1.1220x over previous
import functools

import jax
import jax.numpy as jnp
from jax import lax
from jax.experimental import pallas as pl
from jax.experimental.pallas import tpu as pltpu
from jax.experimental.pallas import tpu_sc as plsc

K = 15
SLOTS = 16
LAM = 1.0


def _mlp_body(x_ref, We1_ref, be1_ref, We2_ref, be2_ref, We3_ref, be3_ref,
              Wd1_ref, bd1_ref, Wd2_ref, bd2_ref, Wd3_ref, bd3_ref,
              z_ref, xr2_ref, zr2_ref, rec_ref):
    pid = pl.program_id(0)
    x = x_ref[...]
    f32 = jnp.float32
    h = jnp.maximum(jnp.dot(x, We1_ref[...], preferred_element_type=f32)
                    + be1_ref[...], 0.0)
    h = jnp.maximum(jnp.dot(h, We2_ref[...], preferred_element_type=f32)
                    + be2_ref[...], 0.0)
    z = jnp.dot(h, We3_ref[...], preferred_element_type=f32) + be3_ref[...]
    h = jnp.maximum(jnp.dot(z, Wd1_ref[...], preferred_element_type=f32)
                    + bd1_ref[...], 0.0)
    h = jnp.maximum(jnp.dot(h, Wd2_ref[...], preferred_element_type=f32)
                    + bd2_ref[...], 0.0)
    x_rec = jnp.dot(h, Wd3_ref[...], preferred_element_type=f32) + bd3_ref[...]
    z_ref[...] = z
    xr2_ref[...] = jnp.sum(x * x, axis=1, keepdims=True)
    zr2_ref[...] = jnp.sum(z * z, axis=1, keepdims=True)

    @pl.when(pid == 0)
    def _():
        rec_ref[...] = jnp.zeros_like(rec_ref)

    diff = x_rec - x
    rec_ref[...] += jnp.sum(diff * diff).reshape(1, 1)


def _knn_body(bm, x_ref, xT_ref, xr2row_ref, vals_ref, idxs_ref, kth_ref):
    pid = pl.program_id(0)
    x = x_ref[...]
    g = jnp.dot(x, xT_ref[...], preferred_element_type=jnp.float32)
    r_blk = jnp.sum(x * x, axis=1, keepdims=True)
    d2 = jnp.maximum(r_blk + xr2row_ref[...] - 2.0 * g, 0.0)
    col = jax.lax.broadcasted_iota(jnp.int32, d2.shape, 1)
    row = jax.lax.broadcasted_iota(jnp.int32, d2.shape, 0) + pid * bm
    work = jnp.where(col == row, jnp.inf, d2)
    m = None
    for t in range(K):
        m = jnp.min(work, axis=1, keepdims=True)
        idx = jnp.min(jnp.where(work == m, col, jnp.int32(2 ** 30)),
                      axis=1, keepdims=True)
        vals_ref[:, t:t + 1] = m
        idxs_ref[:, t:t + 1] = idx
        work = jnp.where(col == idx, jnp.inf, work)
    vals_ref[:, K:K + 1] = jnp.zeros((bm, 1), jnp.float32)
    idxs_ref[:, K:K + 1] = row[:, :1]
    kth_ref[...] = m


def _pair_stage_sc(z, idxs, kth):
    b, lat = z.shape
    nw = 32
    rpw = b // nw
    ch = 8
    nch = rpw // ch
    mesh = plsc.VectorSubcoreMesh(core_axis_name="c", subcore_axis_name="s")
    idxs_flat = idxs.reshape(-1)

    @functools.partial(
        pl.kernel, mesh=mesh,
        compiler_params=pltpu.CompilerParams(needs_layout_passes=False),
        out_type=[
            jax.ShapeDtypeStruct((b * SLOTS,), jnp.float32),
            jax.ShapeDtypeStruct((b * SLOTS,), jnp.float32),
        ],
        scratch_types=[
            pltpu.VMEM((b,), jnp.float32),
            pltpu.VMEM((ch * SLOTS,), jnp.int32),
            pltpu.VMEM((rpw, lat), jnp.float32),
            pltpu.VMEM((ch * SLOTS, lat), jnp.float32),
            pltpu.VMEM((rpw * SLOTS,), jnp.float32),
            pltpu.VMEM((rpw * SLOTS,), jnp.float32),
            pltpu.SemaphoreType.DMA,
        ],
    )
    def k(z_hbm, idxf_hbm, kth_hbm, zd2_hbm, gk_hbm,
          kth_v, idxc_v, zi_v, zj_v, ozd2_v, ogk_v, sem):
        wid = lax.axis_index("c") * 16 + lax.axis_index("s")
        base = wid * rpw
        pltpu.sync_copy(kth_hbm, kth_v)
        pltpu.sync_copy(z_hbm.at[pl.ds(base, rpw)], zi_v)
        lane = lax.iota(jnp.int32, 16)
        for c in range(nch):
            pltpu.sync_copy(
                idxf_hbm.at[pl.ds((base + c * ch) * SLOTS, ch * SLOTS)],
                idxc_v)
            pltpu.async_copy(z_hbm.at[idxc_v], zj_v, sem).wait()

            def rbody(r, carry):
                rr = c * ch + r
                row_ids = r * SLOTS + lane

                def dbody(d, acc):
                    vj = plsc.load_gather(
                        zj_v, [row_ids, jnp.full((16,), d, jnp.int32)])
                    vi = plsc.load_gather(
                        zi_v, [jnp.full((16,), rr, jnp.int32),
                               jnp.full((16,), d, jnp.int32)])
                    dlt = vj - vi
                    return acc + dlt * dlt

                acc = lax.fori_loop(0, lat, dbody,
                                    jnp.zeros((16,), jnp.float32),
                                    unroll=8)
                ozd2_v[pl.ds(rr * SLOTS, SLOTS)] = acc
                idx_row = idxc_v[pl.ds(r * SLOTS, SLOTS)]
                ogk_v[pl.ds(rr * SLOTS, SLOTS)] = plsc.load_gather(
                    kth_v, [idx_row])
                return carry

            lax.fori_loop(0, ch, rbody, 0)
        pltpu.sync_copy(ozd2_v, zd2_hbm.at[pl.ds(base * SLOTS, rpw * SLOTS)])
        pltpu.sync_copy(ogk_v, gk_hbm.at[pl.ds(base * SLOTS, rpw * SLOTS)])

    zd2p, gk = k(z, idxs_flat, kth)
    return zd2p.reshape(b, SLOTS), gk.reshape(b, SLOTS)


def _final_body(b, d_in, vals_ref, zd2p_ref, gk_ref, rec_ref,
                tot_ref, rl_ref, dl_ref):
    xd2 = vals_ref[...]
    zd2 = zd2p_ref[...]
    gk = gk_ref[...]
    slot = jax.lax.broadcasted_iota(jnp.int32, xd2.shape, 1) < K
    kth = xd2[:, K - 1:K]
    xmax = jnp.sqrt(jnp.max(kth))
    zmax = jnp.sqrt(jnp.max(jnp.where(slot, zd2, -jnp.inf)))
    xm = xmax + 1e-8
    zm = zmax + 1e-8
    t = (jnp.sqrt(zd2) / zm - jnp.sqrt(xd2) / xm) ** 2
    s2 = jnp.sum(jnp.where(slot, t, 0.0))
    mut = slot & (xd2 <= gk)
    smut = jnp.sum(jnp.where(mut, t, 0.0))
    nmut = jnp.sum(mut.astype(jnp.float32))
    cnt = 2.0 * b * K - nmut
    dl = ((2.0 * s2 - smut) / cnt).reshape(1, 1)
    rl = rec_ref[...] / (b * d_in)
    dl_ref[...] = dl
    rl_ref[...] = rl
    tot_ref[...] = rl + LAM * dl


def kernel(x, We1, be1, We2, be2, We3, be3, Wd1, bd1, Wd2, bd2, Wd3, bd3):
    b, d_in = x.shape
    h1 = We1.shape[1]
    h2 = We2.shape[1]
    lat = We3.shape[1]
    bm = 256 if b % 256 == 0 else b
    nb = b // bm
    f32 = jnp.float32

    full = lambda shape: pl.BlockSpec(shape, lambda i: (0, 0))
    rowblk = lambda w: pl.BlockSpec((bm, w), lambda i: (i, 0))

    z, xr2, zr2, rec = pl.pallas_call(
        _mlp_body,
        grid=(nb,),
        in_specs=[
            rowblk(d_in),
            full((d_in, h1)), full((1, h1)),
            full((h1, h2)), full((1, h2)),
            full((h2, lat)), full((1, lat)),
            full((lat, h2)), full((1, h2)),
            full((h2, h1)), full((1, h1)),
            full((h1, d_in)), full((1, d_in)),
        ],
        out_specs=[
            rowblk(lat), rowblk(1), rowblk(1),
            pl.BlockSpec((1, 1), lambda i: (0, 0)),
        ],
        out_shape=[
            jax.ShapeDtypeStruct((b, lat), f32),
            jax.ShapeDtypeStruct((b, 1), f32),
            jax.ShapeDtypeStruct((b, 1), f32),
            jax.ShapeDtypeStruct((1, 1), f32),
        ],
    )(x, We1, be1.reshape(1, -1), We2, be2.reshape(1, -1),
      We3, be3.reshape(1, -1), Wd1, bd1.reshape(1, -1),
      Wd2, bd2.reshape(1, -1), Wd3, bd3.reshape(1, -1))

    vals, idxs, kth = pl.pallas_call(
        functools.partial(_knn_body, bm),
        grid=(nb,),
        in_specs=[rowblk(d_in), full((d_in, b)), full((1, b))],
        out_specs=[rowblk(SLOTS), rowblk(SLOTS), rowblk(1)],
        out_shape=[
            jax.ShapeDtypeStruct((b, SLOTS), f32),
            jax.ShapeDtypeStruct((b, SLOTS), jnp.int32),
            jax.ShapeDtypeStruct((b, 1), f32),
        ],
    )(x, x.T, xr2.reshape(1, b))

    del zr2
    zd2p, gk = _pair_stage_sc(z, idxs, kth.reshape(b))

    tot, rl, dl = pl.pallas_call(
        functools.partial(_final_body, b, d_in),
        in_specs=[pl.BlockSpec((b, SLOTS), lambda: (0, 0))] * 3
        + [pl.BlockSpec((1, 1), lambda: (0, 0))],
        out_specs=[pl.BlockSpec((1, 1), lambda: (0, 0))] * 3,
        out_shape=[jax.ShapeDtypeStruct((1, 1), f32)] * 3,
    )(vals, zd2p, gk, rec)

    return (tot[0, 0], rl[0, 0], dl[0, 0])


# speedup vs baseline: 8.2808x; 1.3041x over previous
import functools

import jax
import jax.numpy as jnp
from jax import lax
from jax.experimental import pallas as pl
from jax.experimental.pallas import tpu as pltpu
from jax.experimental.pallas import tpu_sc as plsc

K = 15
SLOTS = 16
LAM = 1.0


def _mlp_body(x_ref, We1_ref, be1_ref, We2_ref, be2_ref, We3_ref, be3_ref,
              Wd1_ref, bd1_ref, Wd2_ref, bd2_ref, Wd3_ref, bd3_ref,
              z_ref, xr2_ref, zr2_ref, rec_ref):
    pid = pl.program_id(0)
    x = x_ref[...]
    f32 = jnp.float32
    h = jnp.maximum(jnp.dot(x, We1_ref[...], preferred_element_type=f32)
                    + be1_ref[...], 0.0)
    h = jnp.maximum(jnp.dot(h, We2_ref[...], preferred_element_type=f32)
                    + be2_ref[...], 0.0)
    z = jnp.dot(h, We3_ref[...], preferred_element_type=f32) + be3_ref[...]
    h = jnp.maximum(jnp.dot(z, Wd1_ref[...], preferred_element_type=f32)
                    + bd1_ref[...], 0.0)
    h = jnp.maximum(jnp.dot(h, Wd2_ref[...], preferred_element_type=f32)
                    + bd2_ref[...], 0.0)
    x_rec = jnp.dot(h, Wd3_ref[...], preferred_element_type=f32) + bd3_ref[...]
    z_ref[...] = z
    xr2_ref[...] = jnp.sum(x * x, axis=1, keepdims=True)
    zr2_ref[...] = jnp.sum(z * z, axis=1, keepdims=True)

    @pl.when(pid == 0)
    def _():
        rec_ref[...] = jnp.zeros_like(rec_ref)

    diff = x_rec - x
    rec_ref[...] += jnp.sum(diff * diff).reshape(1, 1)


def _knn_body(bm, x_ref, xT_ref, xr2row_ref, vals_ref, idxs_ref, kth_ref):
    pid = pl.program_id(0)
    x = x_ref[...]
    g = jnp.dot(x, xT_ref[...], preferred_element_type=jnp.float32)
    r_blk = jnp.sum(x * x, axis=1, keepdims=True)
    d2 = jnp.maximum(r_blk + xr2row_ref[...] - 2.0 * g, 0.0)
    col = jax.lax.broadcasted_iota(jnp.int32, d2.shape, 1)
    row = jax.lax.broadcasted_iota(jnp.int32, d2.shape, 0) + pid * bm
    work = jnp.where(col == row, jnp.inf, d2)
    m = None
    for t in range(K):
        m = jnp.min(work, axis=1, keepdims=True)
        eq = work == m
        idx = jnp.min(jnp.where(eq, col, jnp.int32(2 ** 30)),
                      axis=1, keepdims=True)
        vals_ref[:, t:t + 1] = m
        idxs_ref[:, t:t + 1] = idx
        work = jnp.where(eq, jnp.inf, work)
    vals_ref[:, K:K + 1] = jnp.zeros((bm, 1), jnp.float32)
    idxs_ref[:, K:K + 1] = row[:, :1]
    kth_ref[...] = m


def _pair_stage_sc(z, idxs, kth):
    b, lat = z.shape
    nw = 32
    rpw = b // nw
    ch = 8
    nch = rpw // ch
    mesh = plsc.VectorSubcoreMesh(core_axis_name="c", subcore_axis_name="s")
    idxs_flat = idxs.reshape(-1)

    nd = lat // 16

    @functools.partial(
        pl.kernel, mesh=mesh,
        compiler_params=pltpu.CompilerParams(needs_layout_passes=False),
        out_type=[
            jax.ShapeDtypeStruct((b * SLOTS,), jnp.float32),
            jax.ShapeDtypeStruct((b * SLOTS,), jnp.float32),
        ],
        scratch_types=[
            pltpu.VMEM((b,), jnp.float32),
            pltpu.VMEM((rpw * SLOTS,), jnp.int32),
            pltpu.VMEM((rpw, lat), jnp.float32),
            pltpu.VMEM((ch * SLOTS, lat), jnp.float32),
            pltpu.VMEM((ch * SLOTS, lat), jnp.float32),
            pltpu.VMEM((rpw * SLOTS,), jnp.float32),
            pltpu.VMEM((rpw * SLOTS,), jnp.float32),
            pltpu.SemaphoreType.DMA,
            pltpu.SemaphoreType.DMA,
        ],
    )
    def k(z_hbm, idxf_hbm, kth_hbm, zd2_hbm, gk_hbm,
          kth_v, idx_v, zi2_v, zj0_v, zj1_v, ozd2_v, ogk_v, sem0, sem1):
        wid = lax.axis_index("c") * 16 + lax.axis_index("s")
        base = wid * rpw
        pltpu.sync_copy(kth_hbm, kth_v)
        pltpu.sync_copy(z_hbm.at[pl.ds(base, rpw)], zi2_v)
        pltpu.sync_copy(idxf_hbm.at[pl.ds(base * SLOTS, rpw * SLOTS)], idx_v)
        lane = lax.iota(jnp.int32, 16)
        zjs = (zj0_v, zj1_v)
        sems = (sem0, sem1)

        def start(c):
            return pltpu.async_copy(
                z_hbm.at[idx_v.at[pl.ds(c * ch * SLOTS, ch * SLOTS)]],
                zjs[c % 2], sems[c % 2])

        cps = {0: start(0)}
        for c in range(nch):
            if c + 1 < nch:
                cps[c + 1] = start(c + 1)
            cps.pop(c).wait()
            zj_v = zjs[c % 2]

            def rbody(r, carry):
                rr = c * ch + r
                zic = [zi2_v[rr, pl.ds(dc * 16, 16)] for dc in range(nd)]

                def sbody(s, res):
                    p = r * SLOTS + s
                    acc = jnp.zeros((16,), jnp.float32)
                    for dc in range(nd):
                        vj = zj_v[p, pl.ds(dc * 16, 16)]
                        dlt = vj - zic[dc]
                        acc = acc + dlt * dlt
                    sval = jnp.sum(acc)
                    return jnp.where(lane == s, jnp.full((16,), sval), res)

                res = lax.fori_loop(0, SLOTS, sbody,
                                    jnp.zeros((16,), jnp.float32))
                ozd2_v[pl.ds(rr * SLOTS, SLOTS)] = res
                idx_row = idx_v[pl.ds(rr * SLOTS, SLOTS)]
                ogk_v[pl.ds(rr * SLOTS, SLOTS)] = plsc.load_gather(
                    kth_v, [idx_row])
                return carry

            lax.fori_loop(0, ch, rbody, 0)
        pltpu.sync_copy(ozd2_v, zd2_hbm.at[pl.ds(base * SLOTS, rpw * SLOTS)])
        pltpu.sync_copy(ogk_v, gk_hbm.at[pl.ds(base * SLOTS, rpw * SLOTS)])

    zd2p, gk = k(z, idxs_flat, kth)
    return zd2p.reshape(b, SLOTS), gk.reshape(b, SLOTS)


def _final_body(b, d_in, vals_ref, zd2p_ref, gk_ref, rec_ref,
                tot_ref, rl_ref, dl_ref):
    xd2 = vals_ref[...]
    zd2 = zd2p_ref[...]
    gk = gk_ref[...]
    slot = jax.lax.broadcasted_iota(jnp.int32, xd2.shape, 1) < K
    kth = xd2[:, K - 1:K]
    xmax = jnp.sqrt(jnp.max(kth))
    zmax = jnp.sqrt(jnp.max(jnp.where(slot, zd2, -jnp.inf)))
    xm = xmax + 1e-8
    zm = zmax + 1e-8
    t = (jnp.sqrt(zd2) / zm - jnp.sqrt(xd2) / xm) ** 2
    s2 = jnp.sum(jnp.where(slot, t, 0.0))
    mut = slot & (xd2 <= gk)
    smut = jnp.sum(jnp.where(mut, t, 0.0))
    nmut = jnp.sum(mut.astype(jnp.float32))
    cnt = 2.0 * b * K - nmut
    dl = ((2.0 * s2 - smut) / cnt).reshape(1, 1)
    rl = rec_ref[...] / (b * d_in)
    dl_ref[...] = dl
    rl_ref[...] = rl
    tot_ref[...] = rl + LAM * dl


def kernel(x, We1, be1, We2, be2, We3, be3, Wd1, bd1, Wd2, bd2, Wd3, bd3):
    b, d_in = x.shape
    h1 = We1.shape[1]
    h2 = We2.shape[1]
    lat = We3.shape[1]
    bm = 256 if b % 256 == 0 else b
    nb = b // bm
    f32 = jnp.float32

    full = lambda shape: pl.BlockSpec(shape, lambda i: (0, 0))
    rowblk = lambda w: pl.BlockSpec((bm, w), lambda i: (i, 0))

    z, xr2, zr2, rec = pl.pallas_call(
        _mlp_body,
        grid=(nb,),
        in_specs=[
            rowblk(d_in),
            full((d_in, h1)), full((1, h1)),
            full((h1, h2)), full((1, h2)),
            full((h2, lat)), full((1, lat)),
            full((lat, h2)), full((1, h2)),
            full((h2, h1)), full((1, h1)),
            full((h1, d_in)), full((1, d_in)),
        ],
        out_specs=[
            rowblk(lat), rowblk(1), rowblk(1),
            pl.BlockSpec((1, 1), lambda i: (0, 0)),
        ],
        out_shape=[
            jax.ShapeDtypeStruct((b, lat), f32),
            jax.ShapeDtypeStruct((b, 1), f32),
            jax.ShapeDtypeStruct((b, 1), f32),
            jax.ShapeDtypeStruct((1, 1), f32),
        ],
    )(x, We1, be1.reshape(1, -1), We2, be2.reshape(1, -1),
      We3, be3.reshape(1, -1), Wd1, bd1.reshape(1, -1),
      Wd2, bd2.reshape(1, -1), Wd3, bd3.reshape(1, -1))

    vals, idxs, kth = pl.pallas_call(
        functools.partial(_knn_body, bm),
        grid=(nb,),
        in_specs=[rowblk(d_in), full((d_in, b)), full((1, b))],
        out_specs=[rowblk(SLOTS), rowblk(SLOTS), rowblk(1)],
        out_shape=[
            jax.ShapeDtypeStruct((b, SLOTS), f32),
            jax.ShapeDtypeStruct((b, SLOTS), jnp.int32),
            jax.ShapeDtypeStruct((b, 1), f32),
        ],
    )(x, x.T, xr2.reshape(1, b))

    del zr2
    zd2p, gk = _pair_stage_sc(z, idxs, kth.reshape(b))

    tot, rl, dl = pl.pallas_call(
        functools.partial(_final_body, b, d_in),
        in_specs=[pl.BlockSpec((b, SLOTS), lambda: (0, 0))] * 3
        + [pl.BlockSpec((1, 1), lambda: (0, 0))],
        out_specs=[pl.BlockSpec((1, 1), lambda: (0, 0))] * 3,
        out_shape=[jax.ShapeDtypeStruct((1, 1), f32)] * 3,
    )(vals, zd2p, gk, rec)

    return (tot[0, 0], rl[0, 0], dl[0, 0])


# speedup vs baseline: 9.3137x; 1.1247x over previous
import functools

import jax
import jax.numpy as jnp
from jax import lax
from jax.experimental import pallas as pl
from jax.experimental.pallas import tpu as pltpu
from jax.experimental.pallas import tpu_sc as plsc

K = 15
SLOTS = 16
LAM = 1.0


def _mlp_body(x_ref, We1_ref, be1_ref, We2_ref, be2_ref, We3_ref, be3_ref,
              Wd1_ref, bd1_ref, Wd2_ref, bd2_ref, Wd3_ref, bd3_ref,
              z_ref, xr2_ref, rec_ref):
    pid = pl.program_id(0)
    x = x_ref[...]
    f32 = jnp.float32
    bf = jnp.bfloat16

    def mm(a, w_ref, b_ref):
        return jnp.dot(a.astype(bf), w_ref[...],
                       preferred_element_type=f32) + b_ref[...]

    h = jnp.maximum(mm(x, We1_ref, be1_ref), 0.0)
    h = jnp.maximum(mm(h, We2_ref, be2_ref), 0.0)
    z = mm(h, We3_ref, be3_ref)
    h = jnp.maximum(mm(z, Wd1_ref, bd1_ref), 0.0)
    h = jnp.maximum(mm(h, Wd2_ref, bd2_ref), 0.0)
    x_rec = mm(h, Wd3_ref, bd3_ref)
    z_ref[...] = z
    xr2_ref[...] = jnp.sum(x * x, axis=1, keepdims=True)

    @pl.when(pid == 0)
    def _():
        rec_ref[...] = jnp.zeros_like(rec_ref)

    diff = x_rec - x
    rec_ref[...] += jnp.sum(diff * diff).reshape(1, 1)


def _knn_body(bm, x_ref, xT_ref, xr2row_ref, vals_ref, idxs_ref, kth_ref):
    pid = pl.program_id(0)
    x = x_ref[...]
    g = jnp.dot(x, xT_ref[...], preferred_element_type=jnp.float32)
    r_blk = jnp.sum(x * x, axis=1, keepdims=True)
    d2 = jnp.maximum(r_blk + xr2row_ref[...] - 2.0 * g, 0.0)
    col = jax.lax.broadcasted_iota(jnp.int32, d2.shape, 1)
    row = jax.lax.broadcasted_iota(jnp.int32, d2.shape, 0) + pid * bm
    key = (jax.lax.bitcast_convert_type(d2, jnp.int32) & jnp.int32(~0xFFF)) \
        | col
    bigi = jnp.int32(0x7FFFFFFF)
    work = jnp.where(col == row, bigi, key)
    m = None
    for t in range(K):
        m = jnp.min(work, axis=1, keepdims=True)
        vals_ref[:, t:t + 1] = jax.lax.bitcast_convert_type(
            m & jnp.int32(~0xFFF), jnp.float32)
        idxs_ref[:, t:t + 1] = m & jnp.int32(0xFFF)
        work = jnp.where(work == m, bigi, work)
    vals_ref[:, K:K + 1] = jnp.zeros((bm, 1), jnp.float32)
    idxs_ref[:, K:K + 1] = row[:, :1]
    kth_ref[...] = jax.lax.bitcast_convert_type(
        m & jnp.int32(~0xFFF), jnp.float32)


def _pair_stage_sc(z, idxs, kth):
    b, lat = z.shape
    nw = 32
    rpw = b // nw
    ch = 8
    nch = rpw // ch
    mesh = plsc.VectorSubcoreMesh(core_axis_name="c", subcore_axis_name="s")
    idxs_flat = idxs.reshape(-1)

    nd = lat // 16

    @functools.partial(
        pl.kernel, mesh=mesh,
        compiler_params=pltpu.CompilerParams(needs_layout_passes=False),
        out_type=[
            jax.ShapeDtypeStruct((b * SLOTS,), jnp.float32),
            jax.ShapeDtypeStruct((b * SLOTS,), jnp.float32),
        ],
        scratch_types=[
            pltpu.VMEM((b,), jnp.float32),
            pltpu.VMEM((rpw * SLOTS,), jnp.int32),
            pltpu.VMEM((rpw, lat), jnp.float32),
            pltpu.VMEM((ch * SLOTS, lat), jnp.float32),
            pltpu.VMEM((ch * SLOTS, lat), jnp.float32),
            pltpu.VMEM((rpw * SLOTS,), jnp.float32),
            pltpu.VMEM((rpw * SLOTS,), jnp.float32),
            pltpu.SemaphoreType.DMA,
            pltpu.SemaphoreType.DMA,
        ],
    )
    def k(z_hbm, idxf_hbm, kth_hbm, zd2_hbm, gk_hbm,
          kth_v, idx_v, zi2_v, zj0_v, zj1_v, ozd2_v, ogk_v, sem0, sem1):
        wid = lax.axis_index("c") * 16 + lax.axis_index("s")
        base = wid * rpw
        pltpu.sync_copy(kth_hbm, kth_v)
        pltpu.sync_copy(z_hbm.at[pl.ds(base, rpw)], zi2_v)
        pltpu.sync_copy(idxf_hbm.at[pl.ds(base * SLOTS, rpw * SLOTS)], idx_v)
        lane = lax.iota(jnp.int32, 16)
        zjs = (zj0_v, zj1_v)
        sems = (sem0, sem1)

        def start(c):
            return pltpu.async_copy(
                z_hbm.at[idx_v.at[pl.ds(c * ch * SLOTS, ch * SLOTS)]],
                zjs[c % 2], sems[c % 2])

        cps = {0: start(0)}
        for c in range(nch):
            if c + 1 < nch:
                cps[c + 1] = start(c + 1)
            cps.pop(c).wait()
            zj_v = zjs[c % 2]

            def rbody(r, carry):
                rr = c * ch + r
                zic = [zi2_v[rr, pl.ds(dc * 16, 16)] for dc in range(nd)]

                def sbody(s, res):
                    p = r * SLOTS + s
                    acc = jnp.zeros((16,), jnp.float32)
                    for dc in range(nd):
                        vj = zj_v[p, pl.ds(dc * 16, 16)]
                        dlt = vj - zic[dc]
                        acc = acc + dlt * dlt
                    sval = jnp.sum(acc)
                    return jnp.where(lane == s, jnp.full((16,), sval), res)

                res = lax.fori_loop(0, SLOTS, sbody,
                                    jnp.zeros((16,), jnp.float32))
                ozd2_v[pl.ds(rr * SLOTS, SLOTS)] = res
                idx_row = idx_v[pl.ds(rr * SLOTS, SLOTS)]
                ogk_v[pl.ds(rr * SLOTS, SLOTS)] = plsc.load_gather(
                    kth_v, [idx_row])
                return carry

            lax.fori_loop(0, ch, rbody, 0)
        pltpu.sync_copy(ozd2_v, zd2_hbm.at[pl.ds(base * SLOTS, rpw * SLOTS)])
        pltpu.sync_copy(ogk_v, gk_hbm.at[pl.ds(base * SLOTS, rpw * SLOTS)])

    zd2p, gk = k(z, idxs_flat, kth)
    return zd2p.reshape(b, SLOTS), gk.reshape(b, SLOTS)


def _final_body(b, d_in, vals_ref, zd2p_ref, gk_ref, rec_ref,
                tot_ref, rl_ref, dl_ref):
    xd2 = vals_ref[...]
    zd2 = zd2p_ref[...]
    gk = gk_ref[...]
    slot = jax.lax.broadcasted_iota(jnp.int32, xd2.shape, 1) < K
    kth = xd2[:, K - 1:K]
    xmax = jnp.sqrt(jnp.max(kth))
    zmax = jnp.sqrt(jnp.max(jnp.where(slot, zd2, -jnp.inf)))
    xm = xmax + 1e-8
    zm = zmax + 1e-8
    t = (jnp.sqrt(zd2) / zm - jnp.sqrt(xd2) / xm) ** 2
    s2 = jnp.sum(jnp.where(slot, t, 0.0))
    mut = slot & (xd2 <= gk)
    smut = jnp.sum(jnp.where(mut, t, 0.0))
    nmut = jnp.sum(mut.astype(jnp.float32))
    cnt = 2.0 * b * K - nmut
    dl = ((2.0 * s2 - smut) / cnt).reshape(1, 1)
    rl = rec_ref[...] / (b * d_in)
    dl_ref[...] = dl
    rl_ref[...] = rl
    tot_ref[...] = rl + LAM * dl


def kernel(x, We1, be1, We2, be2, We3, be3, Wd1, bd1, Wd2, bd2, Wd3, bd3):
    b, d_in = x.shape
    h1 = We1.shape[1]
    h2 = We2.shape[1]
    lat = We3.shape[1]
    bm = 256 if b % 256 == 0 else b
    nb = b // bm
    f32 = jnp.float32

    full = lambda shape: pl.BlockSpec(shape, lambda i: (0, 0))
    rowblk = lambda w: pl.BlockSpec((bm, w), lambda i: (i, 0))

    bf = jnp.bfloat16
    z, xr2, rec = pl.pallas_call(
        _mlp_body,
        grid=(nb,),
        in_specs=[
            rowblk(d_in),
            full((d_in, h1)), full((1, h1)),
            full((h1, h2)), full((1, h2)),
            full((h2, lat)), full((1, lat)),
            full((lat, h2)), full((1, h2)),
            full((h2, h1)), full((1, h1)),
            full((h1, d_in)), full((1, d_in)),
        ],
        out_specs=[
            rowblk(lat), rowblk(1),
            pl.BlockSpec((1, 1), lambda i: (0, 0)),
        ],
        out_shape=[
            jax.ShapeDtypeStruct((b, lat), f32),
            jax.ShapeDtypeStruct((b, 1), f32),
            jax.ShapeDtypeStruct((1, 1), f32),
        ],
    )(x, We1.astype(bf), be1.reshape(1, -1), We2.astype(bf),
      be2.reshape(1, -1), We3.astype(bf), be3.reshape(1, -1),
      Wd1.astype(bf), bd1.reshape(1, -1), Wd2.astype(bf),
      bd2.reshape(1, -1), Wd3.astype(bf), bd3.reshape(1, -1))

    vals, idxs, kth = pl.pallas_call(
        functools.partial(_knn_body, bm),
        grid=(nb,),
        in_specs=[rowblk(d_in), full((d_in, b)), full((1, b))],
        out_specs=[rowblk(SLOTS), rowblk(SLOTS), rowblk(1)],
        out_shape=[
            jax.ShapeDtypeStruct((b, SLOTS), f32),
            jax.ShapeDtypeStruct((b, SLOTS), jnp.int32),
            jax.ShapeDtypeStruct((b, 1), f32),
        ],
    )(x, x.T, xr2.reshape(1, b))

    zd2p, gk = _pair_stage_sc(z, idxs, kth.reshape(b))

    tot, rl, dl = pl.pallas_call(
        functools.partial(_final_body, b, d_in),
        in_specs=[pl.BlockSpec((b, SLOTS), lambda: (0, 0))] * 3
        + [pl.BlockSpec((1, 1), lambda: (0, 0))],
        out_specs=[pl.BlockSpec((1, 1), lambda: (0, 0))] * 3,
        out_shape=[jax.ShapeDtypeStruct((1, 1), f32)] * 3,
    )(vals, zd2p, gk, rec)

    return (tot[0, 0], rl[0, 0], dl[0, 0])
